# baseline (speedup 1.0000x reference)
import jax
import jax.numpy as jnp
from jax import lax
from jax.experimental import pallas as pl
from jax.experimental.pallas import tpu as pltpu

_BM = 200
_DEPTH = 3


def _gcn_kernel(x_ref, w_ref, b_ref, adj_ref, o_ref,
                bufs_a, bufs_b, s_ref, sems_a, sems_b):
    n = x_ref.shape[0]
    half = n // (2 * _BM)

    s = jnp.dot(x_ref[...], w_ref[...], preferred_element_type=jnp.float32)
    s_ref[...] = s.astype(jnp.bfloat16)

    def start(idx, slot, bufs, sems):
        pltpu.make_async_copy(
            adj_ref.at[pl.ds(idx * _BM, _BM), :], bufs.at[slot], sems.at[slot]
        ).start()

    for d in range(_DEPTH):
        start(d, d, bufs_a, sems_a)
        start(half + d, d, bufs_b, sems_b)

    def body(i, carry):
        slot = lax.rem(i, _DEPTH)
        pltpu.make_async_copy(
            adj_ref.at[pl.ds(i * _BM, _BM), :],
            bufs_a.at[slot], sems_a.at[slot]).wait()
        pa = jnp.dot(bufs_a[slot].astype(jnp.bfloat16), s_ref[...],
                     preferred_element_type=jnp.float32)
        o_ref[pl.ds(i * _BM, _BM), :] = jnp.maximum(pa + b_ref[...], 0.0)
        j = half + i
        pltpu.make_async_copy(
            adj_ref.at[pl.ds(j * _BM, _BM), :],
            bufs_b.at[slot], sems_b.at[slot]).wait()
        pb = jnp.dot(bufs_b[slot].astype(jnp.bfloat16), s_ref[...],
                     preferred_element_type=jnp.float32)
        o_ref[pl.ds(j * _BM, _BM), :] = jnp.maximum(pb + b_ref[...], 0.0)

        @pl.when(i + _DEPTH < half)
        def _prefetch():
            start(i + _DEPTH, slot, bufs_a, sems_a)
            start(j + _DEPTH, slot, bufs_b, sems_b)

        return carry

    lax.fori_loop(0, half, body, 0)


def kernel(x, adj, W, b):
    n, nfeat = x.shape
    nout = W.shape[1]

    out = pl.pallas_call(
        _gcn_kernel,
        in_specs=[
            pl.BlockSpec(memory_space=pltpu.MemorySpace.VMEM),
            pl.BlockSpec(memory_space=pltpu.MemorySpace.VMEM),
            pl.BlockSpec(memory_space=pltpu.MemorySpace.VMEM),
            pl.BlockSpec(memory_space=pl.ANY),
        ],
        out_specs=pl.BlockSpec(memory_space=pltpu.MemorySpace.VMEM),
        out_shape=jax.ShapeDtypeStruct((n, nout), jnp.float32),
        scratch_shapes=[
            pltpu.VMEM((_DEPTH, _BM, n), jnp.float32),
            pltpu.VMEM((_DEPTH, _BM, n), jnp.float32),
            pltpu.VMEM((n, nout), jnp.bfloat16),
            pltpu.SemaphoreType.DMA((_DEPTH,)),
            pltpu.SemaphoreType.DMA((_DEPTH,)),
        ],
        compiler_params=pltpu.CompilerParams(vmem_limit_bytes=100_000_000),
    )(x, W, b.reshape(1, nout), adj)
    return out


# speedup vs baseline: 1.0203x; 1.0203x over previous
import jax
import jax.numpy as jnp
from jax import lax
from jax.experimental import pallas as pl
from jax.experimental.pallas import tpu as pltpu

_BM = 200
_DEPTH = 5


def _gcn_kernel(x_ref, w_ref, b_ref, adj_ref, o_ref, bufs, s_ref, sems):
    n = x_ref.shape[0]
    nblk = n // _BM
    nouter = nblk // _DEPTH

    s_ref[...] = jnp.dot(x_ref[...], w_ref[...],
                         preferred_element_type=jnp.float32)

    def start(idx, slot):
        pltpu.make_async_copy(
            adj_ref.at[pl.ds(idx * _BM, _BM), :], bufs.at[slot], sems.at[slot]
        ).start()

    for d in range(_DEPTH):
        start(d, d)

    def body(i, carry):
        base = i * _DEPTH
        for s in range(_DEPTH):
            blk = base + s
            pltpu.make_async_copy(
                adj_ref.at[pl.ds(blk * _BM, _BM), :],
                bufs.at[s], sems.at[s]).wait()
            p = jnp.dot(bufs[s], s_ref[...],
                        preferred_element_type=jnp.float32)
            o_ref[pl.ds(blk * _BM, _BM), :] = jnp.maximum(p + b_ref[...], 0.0)

            @pl.when(blk + _DEPTH < nblk)
            def _prefetch():
                start(blk + _DEPTH, s)

        return carry

    lax.fori_loop(0, nouter, body, 0)


def kernel(x, adj, W, b):
    n, nfeat = x.shape
    nout = W.shape[1]

    out = pl.pallas_call(
        _gcn_kernel,
        in_specs=[
            pl.BlockSpec(memory_space=pltpu.MemorySpace.VMEM),
            pl.BlockSpec(memory_space=pltpu.MemorySpace.VMEM),
            pl.BlockSpec(memory_space=pltpu.MemorySpace.VMEM),
            pl.BlockSpec(memory_space=pl.ANY),
        ],
        out_specs=pl.BlockSpec(memory_space=pltpu.MemorySpace.VMEM),
        out_shape=jax.ShapeDtypeStruct((n, nout), jnp.float32),
        scratch_shapes=[
            pltpu.VMEM((_DEPTH, _BM, n), jnp.float32),
            pltpu.VMEM((n, nout), jnp.float32),
            pltpu.SemaphoreType.DMA((_DEPTH,)),
        ],
        compiler_params=pltpu.CompilerParams(vmem_limit_bytes=100_000_000),
    )(x, W, b.reshape(1, nout), adj)
    return out


# speedup vs baseline: 1.0472x; 1.0264x over previous
import jax
import jax.numpy as jnp
from jax.experimental import pallas as pl
from jax.experimental.pallas import tpu as pltpu


def _gcn_kernel(x_ref, w_ref, b_ref, adj_ref, o_ref, s_ref):
    @pl.when(pl.program_id(0) == 0)
    def _support():
        s_ref[...] = jnp.dot(x_ref[...], w_ref[...],
                             preferred_element_type=jnp.float32)

    p = jnp.dot(adj_ref[...], s_ref[...], preferred_element_type=jnp.float32)
    o_ref[...] = jnp.maximum(p + b_ref[...], 0.0)


def kernel(x, adj, W, b):
    n, nfeat = x.shape
    nout = W.shape[1]

    bm = 400
    m_blocks = n // bm

    out = pl.pallas_call(
        _gcn_kernel,
        grid=(m_blocks,),
        in_specs=[
            pl.BlockSpec((n, nfeat), lambda i: (0, 0)),
            pl.BlockSpec((nfeat, nout), lambda i: (0, 0)),
            pl.BlockSpec((1, nout), lambda i: (0, 0)),
            pl.BlockSpec((bm, n), lambda i: (i, 0)),
        ],
        out_specs=pl.BlockSpec((bm, nout), lambda i: (i, 0)),
        out_shape=jax.ShapeDtypeStruct((n, nout), jnp.float32),
        scratch_shapes=[pltpu.VMEM((n, nout), jnp.float32)],
        compiler_params=pltpu.CompilerParams(skip_device_barrier=True),
    )(x, W, b.reshape(1, nout), adj)
    return out
